# baseline (speedup 1.0000x reference)
import dataclasses
import functools

import jax
import jax.numpy as jnp
from jax import lax
from jax.experimental import pallas as pl
from jax.experimental.pallas import tpu as pltpu
from jax.experimental.pallas import tpu_sc as plsc

_DN = (((1,), (1,)), ((), ()))

_NC = 2
_NS = 16
_NEG = float("-inf")


def _pre_body(x_ref, w_ref, y_ref):
    y_ref[...] = jax.lax.dot_general(
        x_ref[...], w_ref[...], _DN, preferred_element_type=jnp.float32)


def _post_body(x_ref, m_ref, y_ref, be_ref, wa_ref, wb_ref, b1_ref,
               g_ref, bt_ref, w2_ref, b2_ref, o_ref):
    m = m_ref[...]
    agg = jnp.where(jnp.isneginf(m), 0.0, m - y_ref[...] + be_ref[...])
    h = (jax.lax.dot_general(x_ref[...], wa_ref[...], _DN,
                             preferred_element_type=jnp.float32)
         + jax.lax.dot_general(agg, wb_ref[...], _DN,
                               preferred_element_type=jnp.float32)
         + b1_ref[...])
    mu = jnp.mean(h, axis=-1, keepdims=True)
    var = jnp.mean((h - mu) ** 2, axis=-1, keepdims=True)
    h = (h - mu) * jax.lax.rsqrt(var + 1e-5) * g_ref[...] + bt_ref[...]
    h = jnp.maximum(h, 0.0)
    o_ref[...] = jax.lax.dot_general(
        h, w2_ref[...], _DN, preferred_element_type=jnp.float32) + b2_ref[...]


def _make_segmax(n, e, d):
    nw = _NC * _NS
    npad = ((n + nw * 16 - 1) // (nw * 16)) * (nw * 16)
    rows = npad // nw
    trash = rows
    chunk = 4000
    nchunk = e // chunk
    ngroup = chunk // 16
    batch = 64
    cap = 8192
    nbm = cap // batch - 1
    depth = 4
    ccap = cap + 16

    mesh = plsc.VectorSubcoreMesh(core_axis_name="c", subcore_axis_name="s")
    cp = pltpu.CompilerParams()
    if "needs_layout_passes" in pltpu.CompilerParams.__dataclass_fields__:
        cp = dataclasses.replace(cp, needs_layout_passes=False)

    @functools.partial(
        pl.kernel,
        out_type=jax.ShapeDtypeStruct((npad, d), jnp.float32),
        mesh=mesh,
        compiler_params=cp,
        scratch_types=[
            pltpu.VMEM((rows + 1, d), jnp.float32),
            pltpu.VMEM((batch, d), jnp.float32),
            pltpu.VMEM((batch, d), jnp.float32),
            pltpu.VMEM((batch, d), jnp.float32),
            pltpu.VMEM((batch, d), jnp.float32),
            pltpu.VMEM((chunk,), jnp.int32),
            pltpu.VMEM((chunk,), jnp.int32),
            pltpu.VMEM((ccap,), jnp.int32),
            pltpu.VMEM((ccap,), jnp.int32),
            pltpu.SemaphoreType.DMA,
            pltpu.SemaphoreType.DMA,
            pltpu.SemaphoreType.DMA,
            pltpu.SemaphoreType.DMA,
            pltpu.SemaphoreType.DMA,
            pltpu.SemaphoreType.DMA,
        ],
    )
    def segmax(y_hbm, e_hbm, out_hbm,
               acc, rv0, rv1, rv2, rv3, eb0, eb1, cbd, cbs,
               es0, es1, gs0, gs1, gs2, gs3):
        wid = lax.axis_index("s") * _NC + lax.axis_index("c")
        lo = wid * rows
        lane = lax.iota(jnp.int32, 16)
        rvs = (rv0, rv1, rv2, rv3)
        gss = (gs0, gs1, gs2, gs3)

        @pl.loop(0, rows + 1)
        def _(i):
            for c in range(d // 16):
                acc[i, pl.ds(c * 16, 16)] = jnp.full((16,), _NEG, jnp.float32)

        @pl.loop(0, ccap, step=16)
        def _(i):
            cbs[pl.ds(i, 16)] = jnp.zeros((16,), jnp.int32)

        def start_chunk(ci, eb, sem):
            cic = jnp.minimum(ci, nchunk - 1)
            pltpu.async_copy(e_hbm.at[pl.ds(cic * chunk, chunk)], eb, sem)

        def wait_chunk(eb, sem):
            pltpu.make_async_copy(e_hbm.at[pl.ds(0, chunk)], eb, sem).wait()

        def scan_chunk(eb, off):
            def g_body(g, off):
                ev = eb[pl.ds(g * 16, 16)]
                dv = jnp.right_shift(ev, 16)
                msk = (dv >= lo) & (dv < lo + rows)
                pos = (off + plsc.cumsum(msk.astype(jnp.int32)) - 1) & (cap - 1)
                plsc.store_scatter(cbd, [pos], dv - lo, mask=msk)
                plsc.store_scatter(cbs, [pos], ev & 0xFFFF, mask=msk)
                return off + plsc.all_reduce_population_count(msk)

            return lax.fori_loop(0, ngroup, g_body, off, unroll=4)

        def _disp4(q, fns):
            def lo2(_):
                return lax.cond(q == 0, fns[0], fns[1], 0)

            def hi2(_):
                return lax.cond(q == 2, fns[2], fns[3], 0)

            return lax.cond(q < 2, lo2, hi2, 0)

        def start_b(b):
            base = (b & nbm) * batch

            def mk(i):
                def f(_):
                    pltpu.async_copy(
                        y_hbm.at[cbs.at[pl.ds(base, batch)]], rvs[i], gss[i])
                    return 0
                return f

            _disp4(b & 3, [mk(0), mk(1), mk(2), mk(3)])

        def accumulate(b, rv, t):
            base = (b & nbm) * batch
            ne = jnp.minimum(batch, t - b * batch)
            ng = (ne + 15) // 16

            @pl.loop(0, ng)
            def _(g):
                dvec = cbd[pl.ds(base + g * 16, 16)]
                for j in range(16):
                    dj = jnp.max(jnp.where(lane == j, dvec, 0))
                    row = g * 16 + j
                    for c in range(d // 16):
                        sl = pl.ds(c * 16, 16)
                        acc[dj, sl] = jnp.maximum(acc[dj, sl], rv[row, sl])

        def proc_b(b, t):
            base = (b & nbm) * batch

            def mk(i):
                def f(_):
                    pltpu.make_async_copy(
                        y_hbm.at[cbs.at[pl.ds(base, batch)]], rvs[i],
                        gss[i]).wait()
                    accumulate(b, rvs[i], t)
                    return 0
                return f

            _disp4(b & 3, [mk(0), mk(1), mk(2), mk(3)])

        def advance(s, p, f_start, f_proc, t):
            for _ in range(depth):
                can = (s < f_start) & (s - p < depth)

                @pl.when(can)
                def _():
                    start_b(s)

                s = jnp.where(can, s + 1, s)

            def body(i, sp):
                s, p = sp
                proc_b(p, t)
                p = p + 1
                can = (s < f_start) & (s - p < depth)

                @pl.when(can)
                def _():
                    start_b(s)

                return (jnp.where(can, s + 1, s), p)

            f_proc = jnp.maximum(f_proc, p)
            return lax.fori_loop(0, f_proc - p, body, (s, p))

        start_chunk(jnp.int32(0), eb0, es0)
        start_chunk(jnp.int32(1), eb1, es1)

        def pair_body(i, carry):
            off, s, p = carry
            c0 = 2 * i
            wait_chunk(eb0, es0)
            off = scan_chunk(eb0, off)
            start_chunk(c0 + 2, eb0, es0)
            t = jnp.max(off)
            f = t // batch
            s, p = advance(s, p, f, f - (depth - 1), t)
            wait_chunk(eb1, es1)
            off = scan_chunk(eb1, off)
            start_chunk(c0 + 3, eb1, es1)
            t = jnp.max(off)
            f = t // batch
            s, p = advance(s, p, f, f - (depth - 1), t)
            return (off, s, p)

        off0 = jnp.zeros((16,), jnp.int32)
        off, s, p = lax.fori_loop(0, nchunk // 2, pair_body,
                                  (off0, jnp.int32(0), jnp.int32(0)))
        wait_chunk(eb0, es0)
        wait_chunk(eb1, es1)

        t = jnp.max(off)
        plsc.store_scatter(cbd, [(t & (cap - 1)) + lane],
                           jnp.full((16,), trash, jnp.int32))
        fc = (t + batch - 1) // batch
        s, p = advance(s, p, fc, fc, t)

        pltpu.sync_copy(acc.at[pl.ds(0, rows)], out_hbm.at[pl.ds(lo, rows)])

    return segmax


def kernel(vertex_features, edge_index, W_edge, b_edge, W_n1, b_n1,
           ln_gamma, ln_beta, W_n2, b_n2):
    n, d = vertex_features.shape
    e = edge_index.shape[1]
    src = edge_index[0].astype(jnp.int32)
    dst = edge_index[1].astype(jnp.int32)
    epk = jnp.left_shift(dst, 16) | src

    y = pl.pallas_call(
        _pre_body,
        out_shape=jax.ShapeDtypeStruct((n, d), jnp.float32),
    )(vertex_features, W_edge)

    mpart = _make_segmax(n, e, d)(y, epk)
    m = mpart[:n]

    W_n1a = W_n1[:, :d]
    W_n1b = W_n1[:, d:]
    out = pl.pallas_call(
        _post_body,
        out_shape=jax.ShapeDtypeStruct((n, d), jnp.float32),
    )(vertex_features, m, y, b_edge.reshape(1, d), W_n1a, W_n1b,
      b_n1.reshape(1, d), ln_gamma.reshape(1, d), ln_beta.reshape(1, d),
      W_n2, b_n2.reshape(1, d))
    return out


# speedup vs baseline: 1.0161x; 1.0161x over previous
import dataclasses
import functools

import jax
import jax.numpy as jnp
from jax import lax
from jax.experimental import pallas as pl
from jax.experimental.pallas import tpu as pltpu
from jax.experimental.pallas import tpu_sc as plsc

_DN = (((1,), (1,)), ((), ()))

_NC = 2
_NS = 16
_NEG = float("-inf")


def _pre_body(x_ref, w_ref, y_ref):
    y_ref[...] = jax.lax.dot_general(
        x_ref[...], w_ref[...], _DN, preferred_element_type=jnp.float32)


def _post_body(x_ref, m_ref, y_ref, be_ref, wa_ref, wb_ref, b1_ref,
               g_ref, bt_ref, w2_ref, b2_ref, o_ref):
    m = m_ref[...]
    agg = jnp.where(jnp.isneginf(m), 0.0, m - y_ref[...] + be_ref[...])
    h = (jax.lax.dot_general(x_ref[...], wa_ref[...], _DN,
                             preferred_element_type=jnp.float32)
         + jax.lax.dot_general(agg, wb_ref[...], _DN,
                               preferred_element_type=jnp.float32)
         + b1_ref[...])
    mu = jnp.mean(h, axis=-1, keepdims=True)
    var = jnp.mean((h - mu) ** 2, axis=-1, keepdims=True)
    h = (h - mu) * jax.lax.rsqrt(var + 1e-5) * g_ref[...] + bt_ref[...]
    h = jnp.maximum(h, 0.0)
    o_ref[...] = jax.lax.dot_general(
        h, w2_ref[...], _DN, preferred_element_type=jnp.float32) + b2_ref[...]


def _make_segmax(n, e, d):
    nw = _NC * _NS
    npad = ((n + nw * 16 - 1) // (nw * 16)) * (nw * 16)
    rows = npad // nw
    trash = rows
    chunk = 4000
    nchunk = e // chunk
    ngroup = chunk // 16
    batch = 64
    cap = 8192
    nbm = cap // batch - 1
    depth = 4
    ccap = cap + 16

    mesh = plsc.VectorSubcoreMesh(core_axis_name="c", subcore_axis_name="s")
    cp = pltpu.CompilerParams()
    if "needs_layout_passes" in pltpu.CompilerParams.__dataclass_fields__:
        cp = dataclasses.replace(cp, needs_layout_passes=False)

    @functools.partial(
        pl.kernel,
        out_type=jax.ShapeDtypeStruct((npad, d), jnp.float32),
        mesh=mesh,
        compiler_params=cp,
        scratch_types=[
            pltpu.VMEM((rows + 1, d), jnp.float32),
            pltpu.VMEM((batch, d), jnp.float32),
            pltpu.VMEM((batch, d), jnp.float32),
            pltpu.VMEM((batch, d), jnp.float32),
            pltpu.VMEM((batch, d), jnp.float32),
            pltpu.VMEM((chunk,), jnp.int32),
            pltpu.VMEM((chunk,), jnp.int32),
            pltpu.VMEM((ccap,), jnp.int32),
            pltpu.VMEM((ccap,), jnp.int32),
            pltpu.SemaphoreType.DMA,
            pltpu.SemaphoreType.DMA,
            pltpu.SemaphoreType.DMA,
            pltpu.SemaphoreType.DMA,
            pltpu.SemaphoreType.DMA,
            pltpu.SemaphoreType.DMA,
        ],
    )
    def segmax(y_hbm, e_hbm, out_hbm,
               acc, rv0, rv1, rv2, rv3, eb0, eb1, cbd, cbs,
               es0, es1, gs0, gs1, gs2, gs3):
        wid = lax.axis_index("s") * _NC + lax.axis_index("c")
        lo = wid * rows
        lane = lax.iota(jnp.int32, 16)
        rvs = (rv0, rv1, rv2, rv3)
        gss = (gs0, gs1, gs2, gs3)

        @pl.loop(0, rows + 1)
        def _(i):
            for c in range(d // 16):
                acc[i, pl.ds(c * 16, 16)] = jnp.full((16,), _NEG, jnp.float32)

        @pl.loop(0, ccap, step=16)
        def _(i):
            cbs[pl.ds(i, 16)] = jnp.zeros((16,), jnp.int32)

        def start_chunk(ci, eb, sem):
            cic = jnp.minimum(ci, nchunk - 1)
            pltpu.async_copy(e_hbm.at[pl.ds(cic * chunk, chunk)], eb, sem)

        def wait_chunk(eb, sem):
            pltpu.make_async_copy(e_hbm.at[pl.ds(0, chunk)], eb, sem).wait()

        lo16 = lo * 65536

        def scan_chunk(eb, off):
            def g_body(g, off):
                ev = eb[pl.ds(g * 16, 16)]
                t = ev - lo16
                msk = t.astype(jnp.uint32) < jnp.uint32(rows * 65536)
                pos = (off + plsc.cumsum(msk.astype(jnp.int32)) - 1) & (cap - 1)
                plsc.store_scatter(cbd, [pos], jnp.right_shift(t, 16), mask=msk)
                plsc.store_scatter(cbs, [pos], ev & 0xFFFF, mask=msk)
                return off + plsc.all_reduce_population_count(msk)

            return lax.fori_loop(0, ngroup, g_body, off, unroll=4)

        def _disp4(q, fns):
            def lo2(_):
                return lax.cond(q == 0, fns[0], fns[1], 0)

            def hi2(_):
                return lax.cond(q == 2, fns[2], fns[3], 0)

            return lax.cond(q < 2, lo2, hi2, 0)

        def start_b(b):
            base = (b & nbm) * batch

            def mk(i):
                def f(_):
                    pltpu.async_copy(
                        y_hbm.at[cbs.at[pl.ds(base, batch)]], rvs[i], gss[i])
                    return 0
                return f

            _disp4(b & 3, [mk(0), mk(1), mk(2), mk(3)])

        def accumulate(b, rv, t):
            base = (b & nbm) * batch
            ne = jnp.minimum(batch, t - b * batch)
            ng = (ne + 15) // 16

            @pl.loop(0, ng)
            def _(g):
                dvec = cbd[pl.ds(base + g * 16, 16)]
                for j in range(16):
                    dj = jnp.max(jnp.where(lane == j, dvec, 0))
                    row = g * 16 + j
                    for c in range(d // 16):
                        sl = pl.ds(c * 16, 16)
                        acc[dj, sl] = jnp.maximum(acc[dj, sl], rv[row, sl])

        def proc_b(b, t):
            base = (b & nbm) * batch

            def mk(i):
                def f(_):
                    pltpu.make_async_copy(
                        y_hbm.at[cbs.at[pl.ds(base, batch)]], rvs[i],
                        gss[i]).wait()
                    accumulate(b, rvs[i], t)
                    return 0
                return f

            _disp4(b & 3, [mk(0), mk(1), mk(2), mk(3)])

        def advance(s, p, f_start, f_proc, t):
            for _ in range(depth):
                can = (s < f_start) & (s - p < depth)

                @pl.when(can)
                def _():
                    start_b(s)

                s = jnp.where(can, s + 1, s)

            def body(i, sp):
                s, p = sp
                proc_b(p, t)
                p = p + 1
                can = (s < f_start) & (s - p < depth)

                @pl.when(can)
                def _():
                    start_b(s)

                return (jnp.where(can, s + 1, s), p)

            f_proc = jnp.maximum(f_proc, p)
            return lax.fori_loop(0, f_proc - p, body, (s, p))

        start_chunk(jnp.int32(0), eb0, es0)
        start_chunk(jnp.int32(1), eb1, es1)

        def pair_body(i, carry):
            off, s, p = carry
            c0 = 2 * i
            wait_chunk(eb0, es0)
            off = scan_chunk(eb0, off)
            start_chunk(c0 + 2, eb0, es0)
            t = jnp.max(off)
            f = t // batch
            s, p = advance(s, p, f, f - (depth - 1), t)
            wait_chunk(eb1, es1)
            off = scan_chunk(eb1, off)
            start_chunk(c0 + 3, eb1, es1)
            t = jnp.max(off)
            f = t // batch
            s, p = advance(s, p, f, f - (depth - 1), t)
            return (off, s, p)

        off0 = jnp.zeros((16,), jnp.int32)
        off, s, p = lax.fori_loop(0, nchunk // 2, pair_body,
                                  (off0, jnp.int32(0), jnp.int32(0)))
        wait_chunk(eb0, es0)
        wait_chunk(eb1, es1)

        t = jnp.max(off)
        plsc.store_scatter(cbd, [(t & (cap - 1)) + lane],
                           jnp.full((16,), trash, jnp.int32))
        fc = (t + batch - 1) // batch
        s, p = advance(s, p, fc, fc, t)

        pltpu.sync_copy(acc.at[pl.ds(0, rows)], out_hbm.at[pl.ds(lo, rows)])

    return segmax


def kernel(vertex_features, edge_index, W_edge, b_edge, W_n1, b_n1,
           ln_gamma, ln_beta, W_n2, b_n2):
    n, d = vertex_features.shape
    e = edge_index.shape[1]
    src = edge_index[0].astype(jnp.int32)
    dst = edge_index[1].astype(jnp.int32)
    epk = jnp.left_shift(dst, 16) | src

    y = pl.pallas_call(
        _pre_body,
        out_shape=jax.ShapeDtypeStruct((n, d), jnp.float32),
    )(vertex_features, W_edge)

    mpart = _make_segmax(n, e, d)(y, epk)
    m = mpart[:n]

    W_n1a = W_n1[:, :d]
    W_n1b = W_n1[:, d:]
    out = pl.pallas_call(
        _post_body,
        out_shape=jax.ShapeDtypeStruct((n, d), jnp.float32),
    )(vertex_features, m, y, b_edge.reshape(1, d), W_n1a, W_n1b,
      b_n1.reshape(1, d), ln_gamma.reshape(1, d), ln_beta.reshape(1, d),
      W_n2, b_n2.reshape(1, d))
    return out
